# Initial kernel scaffold; baseline (speedup 1.0000x reference)
import jax
import jax.numpy as jnp
from jax.experimental import pallas as pl


def kernel(x, adjT, eaT, aeh_all, w_all, avb_all, pool_mat, epool_mat, ea_trunc, ffn_pack):
    raise NotImplementedError("write your pallas kernel here")


# speedup vs baseline: 1.1783x; 1.1783x over previous
import functools

import jax
import jax.numpy as jnp
from jax.experimental import pallas as pl
from jax.experimental.pallas import tpu as pltpu

_LAYER_CFGS = ((2, 16, True), (2, 16, True), (1, 8, False))
_AEH_OFFSETS = (0, 10, 20)
_HMAX = 2
_FFN_DIMS = (8, 4, 6, 3)
_FFN_ROWS = (16, 24, 32)
_BLK = 128


def _block_kernel(x_ref, adj_ref, ea_ref, aeh_ref, w_ref, avb_ref,
                  pool_ref, epool_ref, eat_ref, ffn_ref, o_ref,
                  *, edge_dim):
    mask = adj_ref[...] > 0.0
    neg_big = jnp.float32(-1e30)

    feats = [x_ref[...]]

    for l, (heads, C, concat) in enumerate(_LAYER_CFGS):
        off = _AEH_OFFSETS[l]
        Fc = feats[0].shape[1]
        head_outs = []
        for h in range(heads):
            idx = l * _HMAX + h
            xh = jnp.dot(feats[0], w_ref[idx, 0:Fc, 0:C],
                         preferred_element_type=jnp.float32)
            for k in range(1, len(feats)):
                xh = xh + jnp.dot(feats[k], w_ref[idx, k * Fc:(k + 1) * Fc, 0:C],
                                  preferred_element_type=jnp.float32)

            a_src = jax.lax.dot_general(
                avb_ref[idx, 0:1, 0:C], xh, (((1,), (1,)), ((), ())),
                preferred_element_type=jnp.float32)
            a_dst = jax.lax.dot_general(
                xh, avb_ref[idx, 1:2, 0:C], (((1,), (1,)), ((), ())),
                preferred_element_type=jnp.float32)
            ae = aeh_ref[off + h] * ea_ref[0]
            for d in range(1, edge_dim):
                ae = ae + aeh_ref[off + d * heads + h] * ea_ref[d]

            s = a_dst + a_src + ae
            s = jnp.maximum(s, 0.2 * s)
            s = jnp.where(mask, s, neg_big)
            m = jnp.max(s, axis=-1, keepdims=True)
            p = jnp.where(mask, jnp.exp(s - m), 0.0)
            alpha = p * pl.reciprocal(jnp.sum(p, axis=-1, keepdims=True),
                                      approx=True)
            head_outs.append(jnp.dot(alpha, xh,
                                     preferred_element_type=jnp.float32))

        if concat:
            feats = [jnp.maximum(head_outs[h] + avb_ref[l * _HMAX + h, 2:3, 0:C],
                                 0.0)
                     for h in range(heads)]
        else:
            acc = head_outs[0]
            for t in head_outs[1:]:
                acc = acc + t
            acc = acc * (1.0 / heads) + avb_ref[l * _HMAX, 2:3, 0:C]
            feats = [jnp.maximum(acc, 0.0)]

    h_nodes = feats[0]

    readout = jnp.dot(pool_ref[...], h_nodes,
                      preferred_element_type=jnp.float32)
    og = jnp.dot(epool_ref[...], eat_ref[...],
                 preferred_element_type=jnp.float32)
    sumsq = jnp.sum(og * og, axis=1, keepdims=True)
    og_n = og * jax.lax.rsqrt(jnp.maximum(sumsq, 1e-24))

    d_last, one_gram, d_mid, ncls = _FFN_DIMS
    r_b1, r_w2, r_b2 = _FFN_ROWS
    hid = (jnp.dot(readout, ffn_ref[0:d_last, :],
                   preferred_element_type=jnp.float32)
           + jnp.dot(og_n, ffn_ref[d_last:d_last + one_gram, :],
                     preferred_element_type=jnp.float32)
           + ffn_ref[r_b1:r_b1 + 1, :])
    hid = jnp.maximum(hid, 0.0)
    logits = (jnp.dot(hid, ffn_ref[r_w2:r_w2 + d_mid, 0:ncls],
                      preferred_element_type=jnp.float32)
              + ffn_ref[r_b2:r_b2 + 1, 0:ncls])
    m = jnp.max(logits, axis=1, keepdims=True)
    e = jnp.exp(logits - m)
    o_ref[...] = e / jnp.sum(e, axis=1, keepdims=True)


def kernel(x, adjT, eaT, aeh_all, w_all, avb_all,
           pool_mat, epool_mat, ea_trunc, ffn_pack):
    N = x.shape[0]
    G = pool_mat.shape[0]
    E = epool_mat.shape[1]
    edge_dim = eaT.shape[0]
    ncls = _FFN_DIMS[3]
    blk = _BLK
    nblk = N // blk
    gpb = G // nblk
    epb = E // nblk

    def full(a):
        return pl.BlockSpec(a.shape, lambda i: (0,) * a.ndim)

    specs = [
        pl.BlockSpec((blk, x.shape[1]), lambda i: (i, 0)),
        pl.BlockSpec((blk, blk), lambda i: (i, i)),
        pl.BlockSpec((edge_dim, blk, blk), lambda i: (0, i, i)),
        pl.BlockSpec(memory_space=pltpu.MemorySpace.SMEM),
        full(w_all), full(avb_all),
        pl.BlockSpec((gpb, blk), lambda i: (i, i)),
        pl.BlockSpec((gpb, epb), lambda i: (i, i)),
        pl.BlockSpec((epb, ea_trunc.shape[1]), lambda i: (i, 0)),
        full(ffn_pack),
    ]

    kern = functools.partial(_block_kernel, edge_dim=edge_dim)
    return pl.pallas_call(
        kern,
        out_shape=jax.ShapeDtypeStruct((G, ncls), jnp.float32),
        grid=(nblk,),
        in_specs=specs,
        out_specs=pl.BlockSpec((gpb, ncls), lambda i: (i, 0)),
        compiler_params=pltpu.CompilerParams(
            dimension_semantics=("parallel",),
            vmem_limit_bytes=48 * 1024 * 1024),
    )(x, adjT, eaT, aeh_all, w_all, avb_all,
      pool_mat, epool_mat, ea_trunc, ffn_pack)
